# baseline (speedup 1.0000x reference)
import jax
import jax.numpy as jnp
from jax.experimental import pallas as pl
from jax.experimental.pallas import tpu as pltpu


def _fused_pool_linear_kernel(*refs):
    *x_refs, w_ref, b_ref, o_ref = refs
    tb = o_ref.shape[0]
    tb_sub = tb // len(x_refs)

    stds = []
    means = []
    for x_ref in x_refs:
        x = x_ref[...].astype(jnp.float32)
        n, t = x.shape
        c = n // tb_sub
        s1 = jnp.sum(x, axis=-1, keepdims=True)
        s2 = jnp.sum(x * x, axis=-1, keepdims=True)
        mean = s1 * jnp.float32(1.0 / t)
        ssq = jnp.maximum(s2 - s1 * mean, 0.0)
        std = jnp.sqrt(ssq * jnp.float32(1.0 / (t - 1)))
        stds.append(std.reshape(tb_sub, c))
        means.append(mean.reshape(tb_sub, c))

    std2 = jnp.concatenate(stds, axis=0) if len(stds) > 1 else stds[0]
    mean2 = jnp.concatenate(means, axis=0) if len(means) > 1 else means[0]
    c = std2.shape[1]

    w = w_ref[...].astype(jnp.float32)
    p = w.shape[0] // (2 * c)
    ws = w[0:c]
    wm = w[c:2 * c]
    for k in range(1, p):
        ws = ws + w[k * 2 * c:k * 2 * c + c]
        wm = wm + w[k * 2 * c + c:(k + 1) * 2 * c]

    y = jnp.dot(std2, ws, preferred_element_type=jnp.float32)
    y = y + jnp.dot(mean2, wm, preferred_element_type=jnp.float32)
    o_ref[...] = (y + b_ref[...].astype(jnp.float32)).astype(o_ref.dtype)


def kernel(x, w, b):
    B, C, T = x.shape
    F, O = w.shape

    TB = 32 if B % 32 == 0 else B
    NS = 4 if TB % 4 == 0 else 1
    sub_rows = TB * C // NS

    xf = x.reshape(B * C, T)
    b2 = b.reshape(1, O)

    itemsize = x.dtype.itemsize
    cost = pl.CostEstimate(
        flops=4 * B * C * T + 2 * B * 2 * C * O,
        transcendentals=B * C,
        bytes_accessed=(B * C * T + F * O) * itemsize,
    )

    in_specs = [
        pl.BlockSpec((sub_rows, T), lambda i, k=k: (NS * i + k, 0))
        for k in range(NS)
    ]
    in_specs.append(pl.BlockSpec((F, O), lambda i: (0, 0)))
    in_specs.append(pl.BlockSpec((1, O), lambda i: (0, 0)))

    return pl.pallas_call(
        _fused_pool_linear_kernel,
        out_shape=jax.ShapeDtypeStruct((B, O), x.dtype),
        grid=(B // TB,),
        in_specs=in_specs,
        out_specs=pl.BlockSpec((TB, O), lambda i: (i, 0)),
        compiler_params=pltpu.CompilerParams(
            dimension_semantics=("parallel",),
            vmem_limit_bytes=96 * 1024 * 1024,
        ),
        cost_estimate=cost,
    )(*([xf] * NS), w, b2)


# speedup vs baseline: 1.0009x; 1.0009x over previous
import jax
import jax.numpy as jnp
from jax.experimental import pallas as pl
from jax.experimental.pallas import tpu as pltpu


def _fused_pool_linear_kernel(*refs):
    *x_refs, w_ref, b_ref, o_ref = refs
    tb = o_ref.shape[0]
    tb_sub = tb // len(x_refs)

    stds = []
    means = []
    for x_ref in x_refs:
        x = x_ref[...].astype(jnp.float32)
        n, t = x.shape
        c = n // tb_sub
        mean = jnp.sum(x, axis=-1, keepdims=True) * jnp.float32(1.0 / t)
        d = x - mean
        ssq = jnp.sum(d * d, axis=-1, keepdims=True)
        std = jnp.sqrt(ssq * jnp.float32(1.0 / (t - 1)))
        stds.append(std.reshape(tb_sub, c))
        means.append(mean.reshape(tb_sub, c))

    std2 = jnp.concatenate(stds, axis=0) if len(stds) > 1 else stds[0]
    mean2 = jnp.concatenate(means, axis=0) if len(means) > 1 else means[0]
    c = std2.shape[1]

    w = w_ref[...].astype(jnp.float32)
    p = w.shape[0] // (2 * c)
    ws = w[0:c]
    wm = w[c:2 * c]
    for k in range(1, p):
        ws = ws + w[k * 2 * c:k * 2 * c + c]
        wm = wm + w[k * 2 * c + c:(k + 1) * 2 * c]

    y = jnp.dot(std2, ws, preferred_element_type=jnp.float32)
    y = y + jnp.dot(mean2, wm, preferred_element_type=jnp.float32)
    o_ref[...] = (y + b_ref[...].astype(jnp.float32)).astype(o_ref.dtype)


def kernel(x, w, b):
    B, C, T = x.shape
    F, O = w.shape

    TB = 32 if B % 32 == 0 else B
    NS = 4 if TB % 4 == 0 else 1
    sub_rows = TB * C // NS

    xf = x.reshape(B * C, T)
    b2 = b.reshape(1, O)

    itemsize = x.dtype.itemsize
    cost = pl.CostEstimate(
        flops=4 * B * C * T + 2 * B * 2 * C * O,
        transcendentals=B * C,
        bytes_accessed=(B * C * T + F * O) * itemsize,
    )

    in_specs = [
        pl.BlockSpec((sub_rows, T), lambda i, k=k: (NS * i + k, 0))
        for k in range(NS)
    ]
    in_specs.append(pl.BlockSpec((F, O), lambda i: (0, 0)))
    in_specs.append(pl.BlockSpec((1, O), lambda i: (0, 0)))

    return pl.pallas_call(
        _fused_pool_linear_kernel,
        out_shape=jax.ShapeDtypeStruct((B, O), x.dtype),
        grid=(B // TB,),
        in_specs=in_specs,
        out_specs=pl.BlockSpec((TB, O), lambda i: (i, 0)),
        compiler_params=pltpu.CompilerParams(
            dimension_semantics=("parallel",),
            vmem_limit_bytes=96 * 1024 * 1024,
        ),
        cost_estimate=cost,
    )(*([xf] * NS), w, b2)
